# Initial kernel scaffold; baseline (speedup 1.0000x reference)
import jax
import jax.numpy as jnp
from jax.experimental import pallas as pl


def kernel(x, W, b, time_table, space_table, nan_table):
    raise NotImplementedError("write your pallas kernel here")


# speedup vs baseline: 17.1699x; 17.1699x over previous
import functools

import jax
import jax.numpy as jnp
from jax import lax
from jax.experimental import pallas as pl
from jax.experimental.pallas import tpu as pltpu
from jax.experimental.pallas import tpu_sc as plsc

_NW = 32
_LANES = 16


def _tc_prep(x3, W, b2, time_table, space_pair, nan_table):
    B, T, JD = x3.shape
    D = W.shape[1]

    JDP = ((JD + _LANES - 1) // _LANES) * _LANES

    def body(x_ref, w_ref, b_ref, t_ref, sp_ref, n_ref, r0_ref, r1_ref, m_ref):
        xb = x_ref[...]
        mask = jnp.isnan(xb)
        xc = jnp.where(mask, jnp.float32(0.0), xb)
        lin = lax.dot_general(
            xc.reshape(B * T, JD), w_ref[...],
            (((1,), (0,)), ((), ())),
            preferred_element_type=jnp.float32,
        ).reshape(B, T, D)
        base = lin + (b_ref[0] + n_ref[0])[None, None, :] + t_ref[...][None, :, :]
        r0_ref[...] = base + sp_ref[:, 0, :][None, :, :]
        r1_ref[...] = base + sp_ref[:, 1, :][None, :, :]
        mf = mask.astype(jnp.float32)
        m_ref[...] = jnp.concatenate(
            [mf, jnp.zeros((B, T, JDP - JD), jnp.float32)], axis=2
        )

    return pl.pallas_call(
        body,
        out_shape=(
            jax.ShapeDtypeStruct((B, T, D), jnp.float32),
            jax.ShapeDtypeStruct((B, T, D), jnp.float32),
            jax.ShapeDtypeStruct((B, T, JDP), jnp.float32),
        ),
    )(x3, W, b2, time_table, space_pair, nan_table)


def _sc_expand(row0, row1, maskf, nan_table, n_pairs, jd, d):
    pairs_per_w = n_pairs // _NW
    half = jd // 2
    ncol = d // _LANES
    jdp = maskf.shape[1]
    nchunk = jdp // _LANES
    mesh = plsc.VectorSubcoreMesh(core_axis_name="c", subcore_axis_name="s")

    @functools.partial(
        pl.kernel,
        out_type=jax.ShapeDtypeStruct((n_pairs * jd, d), jnp.float32),
        mesh=mesh,
        compiler_params=pltpu.CompilerParams(use_tc_tiling_on_sc=False),
        scratch_types=[
            pltpu.VMEM((pairs_per_w, d), jnp.float32),
            pltpu.VMEM((pairs_per_w, d), jnp.float32),
            pltpu.VMEM((pairs_per_w, jdp), jnp.float32),
            pltpu.VMEM((2, d), jnp.float32),
            pltpu.VMEM((jd, d), jnp.float32),
        ],
    )
    def k(r0_hbm, r1_hbm, m_hbm, n_hbm, out_hbm, r0v, r1v, mv, nv, ov):
        wid = lax.axis_index("s") * 2 + lax.axis_index("c")
        base = wid * pairs_per_w
        pltpu.sync_copy(r0_hbm.at[pl.ds(base, pairs_per_w)], r0v)
        pltpu.sync_copy(r1_hbm.at[pl.ds(base, pairs_per_w)], r1v)
        pltpu.sync_copy(m_hbm.at[pl.ds(base, pairs_per_w)], mv)
        pltpu.sync_copy(n_hbm, nv)
        delta = [
            nv[1, pl.ds(_LANES * j, _LANES)] - nv[0, pl.ds(_LANES * j, _LANES)]
            for j in range(ncol)
        ]

        def pair_body(p, carry):
            r0 = [r0v[p, pl.ds(_LANES * j, _LANES)] for j in range(ncol)]
            r1 = [r1v[p, pl.ds(_LANES * j, _LANES)] for j in range(ncol)]
            mvecs = [mv[p, pl.ds(_LANES * k, _LANES)] for k in range(nchunk)]
            for s in range(jd):
                src = r0 if s < half else r1
                m = mvecs[s // _LANES][s % _LANES]
                for j in range(ncol):
                    ov[s, pl.ds(_LANES * j, _LANES)] = src[j] + m * delta[j]
            pltpu.sync_copy(ov, out_hbm.at[pl.ds((base + p) * jd, jd)])
            return carry

        lax.fori_loop(0, pairs_per_w, pair_body, 0)

    return k(row0, row1, maskf, nan_table)


def kernel(x, W, b, time_table, space_table, nan_table):
    B, T, J, DX = x.shape
    JD = J * DX
    D = W.shape[1]
    x3 = x.reshape(B, T, JD)
    space_pair = space_table.reshape(T, 2, D)
    row0, row1, maskf = _tc_prep(
        x3, W, b.reshape(1, D), time_table, space_pair, nan_table
    )
    out = _sc_expand(
        row0.reshape(B * T, D),
        row1.reshape(B * T, D),
        maskf.reshape(B * T, maskf.shape[2]),
        nan_table,
        B * T,
        JD,
        D,
    )
    return out.reshape(B, T * JD, D)
